# baseline (speedup 1.0000x reference)
import functools

import jax
import jax.numpy as jnp
from jax import lax
from jax.experimental import pallas as pl
from jax.experimental.pallas import tpu as pltpu
from jax.experimental.pallas import tpu_sc as plsc

H = 1024
I = 4096
E = 8
K = 2
BT = 256
TOK = 512
LANES = 128
NW = 32

C_W0, C_W1, C_E0, C_E1, C_R0, C_R1 = 0, 1, 2, 3, 4, 5


def _gelu(v):
    return 0.5 * v * (1.0 + lax.erf(v * 0.7071067811865476))


def _router_body(x_ref, gw_ref, gb_ref, out_ref, po_ref, te_ref, base_ref):
    t = pl.program_id(0)
    nt = pl.num_programs(0)

    @pl.when(t == 0)
    def _():
        base_ref[...] = jnp.zeros_like(base_ref)

    xb = x_ref[...]
    logits = lax.dot_general(xb, gw_ref[...], (((1,), (1,)), ((), ())),
                             preferred_element_type=jnp.float32)
    lane = lax.broadcasted_iota(jnp.int32, logits.shape, 1)
    valid = lane < E
    sig = jax.nn.sigmoid(logits + gb_ref[...])
    s = jnp.where(valid, sig, -1.0)
    max0 = jnp.max(s, axis=1, keepdims=True)
    idx0 = jnp.min(jnp.where((s == max0) & valid, lane, LANES), axis=1,
                   keepdims=True)
    s1 = jnp.where(lane == idx0, -1.0, s)
    max1 = jnp.max(s1, axis=1, keepdims=True)
    idx1 = jnp.min(jnp.where((s1 == max1) & valid & (lane != idx0), lane,
                             LANES), axis=1, keepdims=True)
    oh0 = (lane == idx0).astype(jnp.float32)
    oh1 = (lane == idx1).astype(jnp.float32)
    m = oh0 + oh1
    r_i = lax.broadcasted_iota(jnp.int32, (TOK, TOK), 0)
    c_i = lax.broadcasted_iota(jnp.int32, (TOK, TOK), 1)
    tril = (r_i > c_i).astype(jnp.float32)
    excl = lax.dot_general(tril, m, (((1,), (0,)), ((), ())),
                           preferred_element_type=jnp.float32)
    base = base_ref[...]
    rank0 = jnp.sum(oh0 * (excl + base), axis=1, keepdims=True)
    rank1 = jnp.sum(oh1 * (excl + base), axis=1, keepdims=True)
    newbase = base + jnp.sum(m, axis=0, keepdims=True)
    base_ref[...] = newbase
    denom = max0 + max1 + 1e-6
    w0 = max0 / denom
    w1 = max1 / denom
    out_ref[...] = (jnp.where(lane == C_W0, w0, 0.0)
                    + jnp.where(lane == C_W1, w1, 0.0)
                    + jnp.where(lane == C_E0, idx0.astype(jnp.float32), 0.0)
                    + jnp.where(lane == C_E1, idx1.astype(jnp.float32), 0.0)
                    + jnp.where(lane == C_R0, rank0, 0.0)
                    + jnp.where(lane == C_R1, rank1, 0.0))

    @pl.when(t == nt - 1)
    def _():
        lane1 = lane[:1, :]
        cnt = newbase[:1, :].astype(jnp.int32)
        pc = ((cnt + BT - 1) // BT) * BT
        pcf = jnp.where(lane1 < E, pc, 0).astype(jnp.float32)
        r2 = lax.broadcasted_iota(jnp.int32, (LANES, LANES), 0)
        c2 = lax.broadcasted_iota(jnp.int32, (LANES, LANES), 1)
        incl = (r2 <= c2).astype(jnp.float32)
        pad_end = lax.dot_general(pcf, incl, (((1,), (0,)), ((), ())),
                                  preferred_element_type=jnp.float32)
        po_ref[...] = jnp.broadcast_to(pad_end - pcf, po_ref.shape)
        ts = (lax.broadcasted_iota(jnp.int32, (LANES, 1), 0)
              * BT).astype(jnp.float32)
        pe_b = jnp.broadcast_to(pad_end, (LANES, LANES))
        a = ((ts >= pe_b) & (c2 < E)).astype(jnp.float32)
        te = jnp.minimum(jnp.sum(a, axis=1, keepdims=True), float(E - 1))
        te_ref[...] = jnp.broadcast_to(te, te_ref.shape)


def _router(x2d, gw_pad, gb_pad, T):
    return pl.pallas_call(
        _router_body,
        grid=(T // TOK,),
        in_specs=[
            pl.BlockSpec((TOK, H), lambda t: (t, 0)),
            pl.BlockSpec((LANES, H), lambda t: (0, 0)),
            pl.BlockSpec((1, LANES), lambda t: (0, 0)),
        ],
        out_specs=[
            pl.BlockSpec((TOK, LANES), lambda t: (t, 0)),
            pl.BlockSpec((8, LANES), lambda t: (0, 0)),
            pl.BlockSpec((LANES, LANES), lambda t: (0, 0)),
        ],
        out_shape=[
            jax.ShapeDtypeStruct((T, LANES), jnp.float32),
            jax.ShapeDtypeStruct((8, LANES), jnp.float32),
            jax.ShapeDtypeStruct((LANES, LANES), jnp.float32),
        ],
        scratch_shapes=[pltpu.VMEM((1, LANES), jnp.float32)],
    )(x2d, gw_pad, gb_pad)


def _cast_body(a_ref, b_ref, c_ref, d_ref, ao_ref, bo_ref, co_ref, do_ref):
    ao_ref[...] = a_ref[...].astype(jnp.bfloat16)
    bo_ref[...] = b_ref[...].astype(jnp.bfloat16)
    co_ref[...] = c_ref[...].astype(jnp.bfloat16)
    do_ref[...] = d_ref[...].astype(jnp.bfloat16)


def _cast_weights(ex_up_w, ex_down_w, sh_up_w, sh_down_w):
    nu = E * I // 2048
    a = ex_up_w.reshape(nu * 2048, H)
    b = ex_down_w.reshape(nu * 2048, H)
    c = sh_up_w.reshape(I, H)
    d = sh_down_w.reshape(I, H)
    sb = I // nu
    outs = pl.pallas_call(
        _cast_body,
        grid=(nu,),
        in_specs=[
            pl.BlockSpec((2048, H), lambda g: (g, 0)),
            pl.BlockSpec((2048, H), lambda g: (g, 0)),
            pl.BlockSpec((sb, H), lambda g: (g, 0)),
            pl.BlockSpec((sb, H), lambda g: (g, 0)),
        ],
        out_specs=[
            pl.BlockSpec((2048, H), lambda g: (g, 0)),
            pl.BlockSpec((2048, H), lambda g: (g, 0)),
            pl.BlockSpec((sb, H), lambda g: (g, 0)),
            pl.BlockSpec((sb, H), lambda g: (g, 0)),
        ],
        out_shape=[
            jax.ShapeDtypeStruct((nu * 2048, H), jnp.bfloat16),
            jax.ShapeDtypeStruct((nu * 2048, H), jnp.bfloat16),
            jax.ShapeDtypeStruct((I, H), jnp.bfloat16),
            jax.ShapeDtypeStruct((I, H), jnp.bfloat16),
        ],
    )(a, b, c, d)
    return (outs[0].reshape(E, I, H), outs[1].reshape(E, H, I),
            outs[2].reshape(I, H), outs[3].reshape(H, I))


def _make_g1(T, P):
    mesh = plsc.VectorSubcoreMesh(core_axis_name="c", subcore_axis_name="s")
    tok_w = T // NW
    CH = 32
    NCH = tok_w // CH

    @functools.partial(
        pl.kernel,
        out_type=jax.ShapeDtypeStruct((P, H), jnp.float32),
        mesh=mesh,
        compiler_params=pltpu.CompilerParams(needs_layout_passes=False),
        scratch_types=[
            pltpu.VMEM((16,), jnp.int32),
            pltpu.VMEM((tok_w, LANES), jnp.float32),
            pltpu.VMEM((2 * NCH, CH), jnp.int32),
            pltpu.VMEM((2, CH, H), jnp.float32),
            pltpu.SemaphoreType.DMA,
        ],
    )
    def g1(po_hbm, pk_hbm, x_hbm, xs_hbm, po_v, pk_v, idx_v, rows_v, sem):
        cid = lax.axis_index("c")
        sid = lax.axis_index("s")
        wid = sid * 2 + cid
        tb = wid * tok_w
        pltpu.sync_copy(po_hbm, po_v)
        pltpu.sync_copy(pk_hbm.at[pl.ds(tb, tok_w)], pk_v)
        ce0 = jnp.full((16,), C_E0, jnp.int32)
        ce1 = jnp.full((16,), C_E1, jnp.int32)
        cr0 = jnp.full((16,), C_R0, jnp.int32)
        cr1 = jnp.full((16,), C_R1, jnp.int32)
        for c in range(NCH):
            for m in range(CH // 16):
                rows16 = lax.broadcasted_iota(jnp.int32, (16,), 0) \
                    + (c * CH + m * 16)
                e0 = plsc.load_gather(pk_v, [rows16, ce0]).astype(jnp.int32)
                r0 = plsc.load_gather(pk_v, [rows16, cr0]).astype(jnp.int32)
                e1 = plsc.load_gather(pk_v, [rows16, ce1]).astype(jnp.int32)
                r1 = plsc.load_gather(pk_v, [rows16, cr1]).astype(jnp.int32)
                sl = pl.ds(m * 16, 16)
                idx_v[c, sl] = plsc.load_gather(po_v, [e0]) + r0
                idx_v[NCH + c, sl] = plsc.load_gather(po_v, [e1]) + r1
        pend = [None, None]
        for c in range(NCH):
            b = c % 2
            if pend[b] is not None:
                pend[b][0].wait()
                pend[b][1].wait()
            pltpu.sync_copy(x_hbm.at[pl.ds(tb + c * CH, CH)], rows_v.at[b])
            d0 = pltpu.async_copy(rows_v.at[b], xs_hbm.at[idx_v.at[c]], sem)
            d1 = pltpu.async_copy(rows_v.at[b], xs_hbm.at[idx_v.at[NCH + c]],
                                  sem)
            pend[b] = (d0, d1)
        for b in range(2):
            if pend[b] is not None:
                pend[b][0].wait()
                pend[b][1].wait()

    return g1


def _expert_body(te_ref, xs_ref, uw_ref, ub_ref, dw_ref, db_ref, ys_ref):
    xb = xs_ref[...].astype(jnp.bfloat16)
    h = lax.dot_general(xb, uw_ref[0], (((1,), (1,)), ((), ())),
                        preferred_element_type=jnp.float32) + ub_ref[0]
    g = _gelu(h).astype(jnp.bfloat16)
    ys_ref[...] = lax.dot_general(g, dw_ref[0], (((1,), (1,)), ((), ())),
                                  preferred_element_type=jnp.float32) + db_ref[0]


def _expert_ffn(xs, te, ex_up_w, ex_up_b, ex_down_w, ex_down_b, P):
    NT = P // BT
    grid_spec = pltpu.PrefetchScalarGridSpec(
        num_scalar_prefetch=1,
        grid=(NT,),
        in_specs=[
            pl.BlockSpec((BT, H), lambda t, te: (t, 0)),
            pl.BlockSpec((1, I, H), lambda t, te: (te[t], 0, 0)),
            pl.BlockSpec((1, 1, I), lambda t, te: (te[t], 0, 0)),
            pl.BlockSpec((1, H, I), lambda t, te: (te[t], 0, 0)),
            pl.BlockSpec((1, 1, H), lambda t, te: (te[t], 0, 0)),
        ],
        out_specs=pl.BlockSpec((BT, H), lambda t, te: (t, 0)),
    )
    return pl.pallas_call(
        _expert_body,
        grid_spec=grid_spec,
        out_shape=jax.ShapeDtypeStruct((P, H), jnp.float32),
    )(te, xs, ex_up_w, ex_up_b.reshape(E, 1, I), ex_down_w,
      ex_down_b.reshape(E, 1, H))


def _make_g2(T, P):
    mesh = plsc.VectorSubcoreMesh(core_axis_name="c", subcore_axis_name="s")
    tok_w = T // NW
    GC = 32
    NC2 = tok_w // GC

    @functools.partial(
        pl.kernel,
        out_type=(jax.ShapeDtypeStruct((T, H), jnp.float32),
                  jax.ShapeDtypeStruct((T, H), jnp.float32)),
        mesh=mesh,
        compiler_params=pltpu.CompilerParams(needs_layout_passes=False),
        scratch_types=[
            pltpu.VMEM((16,), jnp.int32),
            pltpu.VMEM((tok_w, LANES), jnp.float32),
            pltpu.VMEM((2 * NC2, GC), jnp.int32),
            pltpu.VMEM((2, GC, H), jnp.float32),
            pltpu.SemaphoreType.DMA,
            pltpu.SemaphoreType.DMA,
        ],
    )
    def g2(po_hbm, pk_hbm, ys_hbm, a_hbm, b_hbm,
           po_v, pk_v, idx_v, rows_v, sem, sem2):
        cid = lax.axis_index("c")
        sid = lax.axis_index("s")
        wid = sid * 2 + cid
        tb = wid * tok_w
        pltpu.sync_copy(po_hbm, po_v)
        pltpu.sync_copy(pk_hbm.at[pl.ds(tb, tok_w)], pk_v)
        ce0 = jnp.full((16,), C_E0, jnp.int32)
        ce1 = jnp.full((16,), C_E1, jnp.int32)
        cr0 = jnp.full((16,), C_R0, jnp.int32)
        cr1 = jnp.full((16,), C_R1, jnp.int32)
        for c in range(NC2):
            for m in range(GC // 16):
                rows16 = lax.broadcasted_iota(jnp.int32, (16,), 0) \
                    + (c * GC + m * 16)
                e0 = plsc.load_gather(pk_v, [rows16, ce0]).astype(jnp.int32)
                r0 = plsc.load_gather(pk_v, [rows16, cr0]).astype(jnp.int32)
                e1 = plsc.load_gather(pk_v, [rows16, ce1]).astype(jnp.int32)
                r1 = plsc.load_gather(pk_v, [rows16, cr1]).astype(jnp.int32)
                sl = pl.ds(m * 16, 16)
                idx_v[c, sl] = plsc.load_gather(po_v, [e0]) + r0
                idx_v[NC2 + c, sl] = plsc.load_gather(po_v, [e1]) + r1
        pend = [None, None]
        for side, out_hbm in ((0, a_hbm), (1, b_hbm)):
            for c in range(NC2):
                b = (side * NC2 + c) % 2
                if pend[b] is not None:
                    pend[b].wait()
                pltpu.async_copy(ys_hbm.at[idx_v.at[side * NC2 + c]],
                                 rows_v.at[b], sem).wait()
                pend[b] = pltpu.async_copy(
                    rows_v.at[b], out_hbm.at[pl.ds(tb + c * GC, GC)], sem2)
        for b in range(2):
            if pend[b] is not None:
                pend[b].wait()

    return g2


def _shared_body(x_ref, uw_ref, ub_ref, dw_ref, db_ref, out_ref):
    xb = x_ref[...].astype(jnp.bfloat16)
    h = lax.dot_general(xb, uw_ref[...], (((1,), (1,)), ((), ())),
                        preferred_element_type=jnp.float32) + ub_ref[...]
    g = _gelu(h).astype(jnp.bfloat16)
    out_ref[...] = lax.dot_general(g, dw_ref[...], (((1,), (1,)), ((), ())),
                                   preferred_element_type=jnp.float32) + db_ref[...]


def _shared(x2d, sh_up_w, sh_up_b, sh_down_w, sh_down_b, T):
    return pl.pallas_call(
        _shared_body,
        grid=(T // TOK,),
        in_specs=[
            pl.BlockSpec((TOK, H), lambda t: (t, 0)),
            pl.BlockSpec((I, H), lambda t: (0, 0)),
            pl.BlockSpec((1, I), lambda t: (0, 0)),
            pl.BlockSpec((H, I), lambda t: (0, 0)),
            pl.BlockSpec((1, H), lambda t: (0, 0)),
        ],
        out_specs=pl.BlockSpec((TOK, H), lambda t: (t, 0)),
        out_shape=jax.ShapeDtypeStruct((T, H), jnp.float32),
    )(x2d, sh_up_w, sh_up_b, sh_down_w, sh_down_b)


def _combine_body(sh_ref, pk_ref, a_ref, b_ref, out_ref):
    pk = pk_ref[...]
    lane = lax.broadcasted_iota(jnp.int32, pk.shape, 1)
    w0 = jnp.sum(jnp.where(lane == C_W0, pk, 0.0), axis=1, keepdims=True)
    w1 = jnp.sum(jnp.where(lane == C_W1, pk, 0.0), axis=1, keepdims=True)
    t = sh_ref[...] + w0 * a_ref[...] + w1 * b_ref[...]
    mo = 0.1 * t
    out_ref[...] = mo / (jnp.max(jnp.abs(mo), axis=1, keepdims=True) + 1e-6)


def _combine(sh, packed, a, b, T):
    return pl.pallas_call(
        _combine_body,
        grid=(T // TOK,),
        in_specs=[
            pl.BlockSpec((TOK, H), lambda t: (t, 0)),
            pl.BlockSpec((TOK, LANES), lambda t: (t, 0)),
            pl.BlockSpec((TOK, H), lambda t: (t, 0)),
            pl.BlockSpec((TOK, H), lambda t: (t, 0)),
        ],
        out_specs=pl.BlockSpec((TOK, H), lambda t: (t, 0)),
        out_shape=jax.ShapeDtypeStruct((T, H), jnp.float32),
    )(sh, packed, a, b)


def kernel(x, gate_w, moe_bias, sh_up_w, sh_up_b, sh_down_w, sh_down_b,
           ex_up_w, ex_up_b, ex_down_w, ex_down_b):
    B, S, _ = x.shape
    T = B * S
    P = ((T * K + E * (BT - 1)) + BT - 1) // BT * BT
    NT = P // BT
    x2d = x.reshape(T, H)

    gw_pad = jnp.zeros((LANES, H), jnp.float32).at[:E].set(gate_w)
    gb_pad = jnp.zeros((1, LANES), jnp.float32).at[0, :E].set(moe_bias)

    ex_up_bf, ex_down_bf, sh_up_bf, sh_down_bf = _cast_weights(
        ex_up_w, ex_down_w, sh_up_w, sh_down_w)

    packed, po, teo = _router(x2d, gw_pad, gb_pad, T)
    pad_off16 = po[0, :16].astype(jnp.int32)
    te = teo[:NT, 0].astype(jnp.int32)

    sh = _shared(x2d, sh_up_bf, sh_up_b.reshape(1, I), sh_down_bf,
                 sh_down_b.reshape(1, H), T)
    xs = _make_g1(T, P)(pad_off16, packed, x2d)
    ys = _expert_ffn(xs, te, ex_up_bf, ex_up_b, ex_down_bf, ex_down_b, P)
    a, b = _make_g2(T, P)(pad_off16, packed, ys)
    out = _combine(sh, packed, a, b, T)
    return out.reshape(B, S, H)


# speedup vs baseline: 1.6351x; 1.6351x over previous
import functools

import jax
import jax.numpy as jnp
from jax import lax
from jax.experimental import pallas as pl
from jax.experimental.pallas import tpu as pltpu
from jax.experimental.pallas import tpu_sc as plsc

H = 1024
I = 4096
E = 8
K = 2
BT = 256
TOK = 512
LANES = 128
NW = 32

C_W0, C_W1, C_E0, C_E1, C_R0, C_R1 = 0, 1, 2, 3, 4, 5


def _gelu(v):
    return 0.5 * v * (1.0 + lax.erf(v * 0.7071067811865476))


def _router_body(x_ref, gw_ref, gb_ref, out_ref, po_ref, te_ref, base_ref):
    t = pl.program_id(0)
    nt = pl.num_programs(0)

    @pl.when(t == 0)
    def _():
        base_ref[...] = jnp.zeros_like(base_ref)

    xb = x_ref[...]
    logits = lax.dot_general(xb, gw_ref[...], (((1,), (1,)), ((), ())),
                             preferred_element_type=jnp.float32)
    lane = lax.broadcasted_iota(jnp.int32, logits.shape, 1)
    valid = lane < E
    sig = jax.nn.sigmoid(logits + gb_ref[...])
    s = jnp.where(valid, sig, -1.0)
    max0 = jnp.max(s, axis=1, keepdims=True)
    idx0 = jnp.min(jnp.where((s == max0) & valid, lane, LANES), axis=1,
                   keepdims=True)
    s1 = jnp.where(lane == idx0, -1.0, s)
    max1 = jnp.max(s1, axis=1, keepdims=True)
    idx1 = jnp.min(jnp.where((s1 == max1) & valid & (lane != idx0), lane,
                             LANES), axis=1, keepdims=True)
    oh0 = (lane == idx0).astype(jnp.float32)
    oh1 = (lane == idx1).astype(jnp.float32)
    m = oh0 + oh1
    r_i = lax.broadcasted_iota(jnp.int32, (TOK, TOK), 0)
    c_i = lax.broadcasted_iota(jnp.int32, (TOK, TOK), 1)
    tril = (r_i > c_i).astype(jnp.float32)
    excl = lax.dot_general(tril, m, (((1,), (0,)), ((), ())),
                           preferred_element_type=jnp.float32)
    base = base_ref[...]
    rank0 = jnp.sum(oh0 * (excl + base), axis=1, keepdims=True)
    rank1 = jnp.sum(oh1 * (excl + base), axis=1, keepdims=True)
    newbase = base + jnp.sum(m, axis=0, keepdims=True)
    base_ref[...] = newbase
    denom = max0 + max1 + 1e-6
    w0 = max0 / denom
    w1 = max1 / denom
    out_ref[...] = (jnp.where(lane == C_W0, w0, 0.0)
                    + jnp.where(lane == C_W1, w1, 0.0)
                    + jnp.where(lane == C_E0, idx0.astype(jnp.float32), 0.0)
                    + jnp.where(lane == C_E1, idx1.astype(jnp.float32), 0.0)
                    + jnp.where(lane == C_R0, rank0, 0.0)
                    + jnp.where(lane == C_R1, rank1, 0.0))

    @pl.when(t == nt - 1)
    def _():
        lane1 = lane[:1, :]
        cnt = newbase[:1, :].astype(jnp.int32)
        pc = ((cnt + BT - 1) // BT) * BT
        pcf = jnp.where(lane1 < E, pc, 0).astype(jnp.float32)
        r2 = lax.broadcasted_iota(jnp.int32, (LANES, LANES), 0)
        c2 = lax.broadcasted_iota(jnp.int32, (LANES, LANES), 1)
        incl = (r2 <= c2).astype(jnp.float32)
        pad_end = lax.dot_general(pcf, incl, (((1,), (0,)), ((), ())),
                                  preferred_element_type=jnp.float32)
        po_ref[...] = jnp.broadcast_to(pad_end - pcf, po_ref.shape)
        ts = (lax.broadcasted_iota(jnp.int32, (LANES, 1), 0)
              * BT).astype(jnp.float32)
        pe_b = jnp.broadcast_to(pad_end, (LANES, LANES))
        a = ((ts >= pe_b) & (c2 < E)).astype(jnp.float32)
        te = jnp.minimum(jnp.sum(a, axis=1, keepdims=True), float(E - 1))
        total = jnp.max(pad_end, axis=1, keepdims=True)
        te = jnp.where(ts < total, te, -1.0)
        te_ref[...] = jnp.broadcast_to(te, te_ref.shape)


def _router(x2d, gw_pad, gb_pad, T):
    return pl.pallas_call(
        _router_body,
        grid=(T // TOK,),
        in_specs=[
            pl.BlockSpec((TOK, H), lambda t: (t, 0)),
            pl.BlockSpec((LANES, H), lambda t: (0, 0)),
            pl.BlockSpec((1, LANES), lambda t: (0, 0)),
        ],
        out_specs=[
            pl.BlockSpec((TOK, LANES), lambda t: (t, 0)),
            pl.BlockSpec((8, LANES), lambda t: (0, 0)),
            pl.BlockSpec((LANES, LANES), lambda t: (0, 0)),
        ],
        out_shape=[
            jax.ShapeDtypeStruct((T, LANES), jnp.float32),
            jax.ShapeDtypeStruct((8, LANES), jnp.float32),
            jax.ShapeDtypeStruct((LANES, LANES), jnp.float32),
        ],
        scratch_shapes=[pltpu.VMEM((1, LANES), jnp.float32)],
    )(x2d, gw_pad, gb_pad)


def _cast_body(a_ref, b_ref, c_ref, d_ref, ao_ref, bo_ref, co_ref, do_ref):
    ao_ref[...] = a_ref[...].astype(jnp.bfloat16)
    bo_ref[...] = b_ref[...].astype(jnp.bfloat16)
    co_ref[...] = c_ref[...].astype(jnp.bfloat16)
    do_ref[...] = d_ref[...].astype(jnp.bfloat16)


def _cast_weights(ex_up_w, ex_down_w, sh_up_w, sh_down_w):
    ng = 16
    a = ex_up_w.reshape(E * I, H)
    b = ex_down_w.reshape(E * H, I)
    au, bu, cu, du = E * I // ng, E * H // ng, I // ng, H // ng
    outs = pl.pallas_call(
        _cast_body,
        grid=(ng,),
        in_specs=[
            pl.BlockSpec((au, H), lambda g: (g, 0)),
            pl.BlockSpec((bu, I), lambda g: (g, 0)),
            pl.BlockSpec((cu, H), lambda g: (g, 0)),
            pl.BlockSpec((du, I), lambda g: (g, 0)),
        ],
        out_specs=[
            pl.BlockSpec((au, H), lambda g: (g, 0)),
            pl.BlockSpec((bu, I), lambda g: (g, 0)),
            pl.BlockSpec((cu, H), lambda g: (g, 0)),
            pl.BlockSpec((du, I), lambda g: (g, 0)),
        ],
        out_shape=[
            jax.ShapeDtypeStruct((E * I, H), jnp.bfloat16),
            jax.ShapeDtypeStruct((E * H, I), jnp.bfloat16),
            jax.ShapeDtypeStruct((I, H), jnp.bfloat16),
            jax.ShapeDtypeStruct((H, I), jnp.bfloat16),
        ],
    )(a, b, sh_up_w, sh_down_w)
    return (outs[0].reshape(E, I, H), outs[1].reshape(E, H, I),
            outs[2], outs[3])


def _make_g1(T, P):
    mesh = plsc.VectorSubcoreMesh(core_axis_name="c", subcore_axis_name="s")
    tok_w = T // NW
    CH = 32
    NCH = tok_w // CH

    @functools.partial(
        pl.kernel,
        out_type=jax.ShapeDtypeStruct((P, H), jnp.float32),
        mesh=mesh,
        compiler_params=pltpu.CompilerParams(needs_layout_passes=False),
        scratch_types=[
            pltpu.VMEM((16,), jnp.int32),
            pltpu.VMEM((tok_w, LANES), jnp.float32),
            pltpu.VMEM((2 * NCH, CH), jnp.int32),
            pltpu.VMEM((2, CH, H), jnp.float32),
            pltpu.SemaphoreType.DMA,
        ],
    )
    def g1(po_hbm, pk_hbm, x_hbm, xs_hbm, po_v, pk_v, idx_v, rows_v, sem):
        cid = lax.axis_index("c")
        sid = lax.axis_index("s")
        wid = sid * 2 + cid
        tb = wid * tok_w
        pltpu.sync_copy(po_hbm, po_v)
        pltpu.sync_copy(pk_hbm.at[pl.ds(tb, tok_w)], pk_v)
        ce0 = jnp.full((16,), C_E0, jnp.int32)
        ce1 = jnp.full((16,), C_E1, jnp.int32)
        cr0 = jnp.full((16,), C_R0, jnp.int32)
        cr1 = jnp.full((16,), C_R1, jnp.int32)
        for c in range(NCH):
            for m in range(CH // 16):
                rows16 = lax.broadcasted_iota(jnp.int32, (16,), 0) \
                    + (c * CH + m * 16)
                e0 = plsc.load_gather(pk_v, [rows16, ce0]).astype(jnp.int32)
                r0 = plsc.load_gather(pk_v, [rows16, cr0]).astype(jnp.int32)
                e1 = plsc.load_gather(pk_v, [rows16, ce1]).astype(jnp.int32)
                r1 = plsc.load_gather(pk_v, [rows16, cr1]).astype(jnp.int32)
                sl = pl.ds(m * 16, 16)
                idx_v[c, sl] = plsc.load_gather(po_v, [e0]) + r0
                idx_v[NCH + c, sl] = plsc.load_gather(po_v, [e1]) + r1
        pend = [None, None]
        for c in range(NCH):
            b = c % 2
            if pend[b] is not None:
                pend[b][0].wait()
                pend[b][1].wait()
            pltpu.sync_copy(x_hbm.at[pl.ds(tb + c * CH, CH)], rows_v.at[b])
            d0 = pltpu.async_copy(rows_v.at[b], xs_hbm.at[idx_v.at[c]], sem)
            d1 = pltpu.async_copy(rows_v.at[b], xs_hbm.at[idx_v.at[NCH + c]],
                                  sem)
            pend[b] = (d0, d1)
        for b in range(2):
            if pend[b] is not None:
                pend[b][0].wait()
                pend[b][1].wait()

    return g1


def _expert_body(te_ref, xs_ref, uw_ref, ub_ref, dw_ref, db_ref, ys_ref):
    t = pl.program_id(0)

    @pl.when(te_ref[t] >= 0)
    def _():
        xb = xs_ref[...].astype(jnp.bfloat16)
        h = lax.dot_general(xb, uw_ref[0], (((1,), (1,)), ((), ())),
                            preferred_element_type=jnp.float32) + ub_ref[0]
        g = _gelu(h).astype(jnp.bfloat16)
        ys_ref[...] = lax.dot_general(g, dw_ref[0], (((1,), (1,)), ((), ())),
                                      preferred_element_type=jnp.float32) \
            + db_ref[0]


def _expert_ffn(xs, te, ex_up_w, ex_up_b, ex_down_w, ex_down_b, P):
    NT = P // BT
    grid_spec = pltpu.PrefetchScalarGridSpec(
        num_scalar_prefetch=1,
        grid=(NT,),
        in_specs=[
            pl.BlockSpec((BT, H), lambda t, te: (t, 0)),
            pl.BlockSpec((1, I, H), lambda t, te: (jnp.maximum(te[t], 0), 0, 0)),
            pl.BlockSpec((1, 1, I), lambda t, te: (jnp.maximum(te[t], 0), 0, 0)),
            pl.BlockSpec((1, H, I), lambda t, te: (jnp.maximum(te[t], 0), 0, 0)),
            pl.BlockSpec((1, 1, H), lambda t, te: (jnp.maximum(te[t], 0), 0, 0)),
        ],
        out_specs=pl.BlockSpec((BT, H), lambda t, te: (t, 0)),
    )
    return pl.pallas_call(
        _expert_body,
        grid_spec=grid_spec,
        out_shape=jax.ShapeDtypeStruct((P, H), jnp.float32),
    )(te, xs, ex_up_w, ex_up_b.reshape(E, 1, I), ex_down_w,
      ex_down_b.reshape(E, 1, H))


def _make_g2(T, P):
    mesh = plsc.VectorSubcoreMesh(core_axis_name="c", subcore_axis_name="s")
    tok_w = T // NW
    GC = 32
    NC2 = tok_w // GC

    @functools.partial(
        pl.kernel,
        out_type=(jax.ShapeDtypeStruct((T, H), jnp.float32),
                  jax.ShapeDtypeStruct((T, H), jnp.float32)),
        mesh=mesh,
        compiler_params=pltpu.CompilerParams(needs_layout_passes=False),
        scratch_types=[
            pltpu.VMEM((16,), jnp.int32),
            pltpu.VMEM((tok_w, LANES), jnp.float32),
            pltpu.VMEM((2 * NC2, GC), jnp.int32),
            pltpu.VMEM((2, GC, H), jnp.float32),
            pltpu.SemaphoreType.DMA,
            pltpu.SemaphoreType.DMA,
        ],
    )
    def g2(po_hbm, pk_hbm, ys_hbm, a_hbm, b_hbm,
           po_v, pk_v, idx_v, rows_v, sem, sem2):
        cid = lax.axis_index("c")
        sid = lax.axis_index("s")
        wid = sid * 2 + cid
        tb = wid * tok_w
        pltpu.sync_copy(po_hbm, po_v)
        pltpu.sync_copy(pk_hbm.at[pl.ds(tb, tok_w)], pk_v)
        ce0 = jnp.full((16,), C_E0, jnp.int32)
        ce1 = jnp.full((16,), C_E1, jnp.int32)
        cr0 = jnp.full((16,), C_R0, jnp.int32)
        cr1 = jnp.full((16,), C_R1, jnp.int32)
        for c in range(NC2):
            for m in range(GC // 16):
                rows16 = lax.broadcasted_iota(jnp.int32, (16,), 0) \
                    + (c * GC + m * 16)
                e0 = plsc.load_gather(pk_v, [rows16, ce0]).astype(jnp.int32)
                r0 = plsc.load_gather(pk_v, [rows16, cr0]).astype(jnp.int32)
                e1 = plsc.load_gather(pk_v, [rows16, ce1]).astype(jnp.int32)
                r1 = plsc.load_gather(pk_v, [rows16, cr1]).astype(jnp.int32)
                sl = pl.ds(m * 16, 16)
                idx_v[c, sl] = plsc.load_gather(po_v, [e0]) + r0
                idx_v[NC2 + c, sl] = plsc.load_gather(po_v, [e1]) + r1
        pend = [None, None]
        for side, out_hbm in ((0, a_hbm), (1, b_hbm)):
            for c in range(NC2):
                b = (side * NC2 + c) % 2
                if pend[b] is not None:
                    pend[b].wait()
                pltpu.async_copy(ys_hbm.at[idx_v.at[side * NC2 + c]],
                                 rows_v.at[b], sem).wait()
                pend[b] = pltpu.async_copy(
                    rows_v.at[b], out_hbm.at[pl.ds(tb + c * GC, GC)], sem2)
        for b in range(2):
            if pend[b] is not None:
                pend[b].wait()

    return g2


def _shared_body(x_ref, uw_ref, ub_ref, dw_ref, db_ref, out_ref):
    xb = x_ref[...].astype(jnp.bfloat16)
    h = lax.dot_general(xb, uw_ref[...], (((1,), (1,)), ((), ())),
                        preferred_element_type=jnp.float32) + ub_ref[...]
    g = _gelu(h).astype(jnp.bfloat16)
    out_ref[...] = lax.dot_general(g, dw_ref[...], (((1,), (1,)), ((), ())),
                                   preferred_element_type=jnp.float32) + db_ref[...]


def _shared(x2d, sh_up_w, sh_up_b, sh_down_w, sh_down_b, T):
    return pl.pallas_call(
        _shared_body,
        grid=(T // TOK,),
        in_specs=[
            pl.BlockSpec((TOK, H), lambda t: (t, 0)),
            pl.BlockSpec((I, H), lambda t: (0, 0)),
            pl.BlockSpec((1, I), lambda t: (0, 0)),
            pl.BlockSpec((H, I), lambda t: (0, 0)),
            pl.BlockSpec((1, H), lambda t: (0, 0)),
        ],
        out_specs=pl.BlockSpec((TOK, H), lambda t: (t, 0)),
        out_shape=jax.ShapeDtypeStruct((T, H), jnp.float32),
    )(x2d, sh_up_w, sh_up_b, sh_down_w, sh_down_b)


def _combine_body(sh_ref, pk_ref, a_ref, b_ref, out_ref):
    pk = pk_ref[...]
    lane = lax.broadcasted_iota(jnp.int32, pk.shape, 1)
    w0 = jnp.sum(jnp.where(lane == C_W0, pk, 0.0), axis=1, keepdims=True)
    w1 = jnp.sum(jnp.where(lane == C_W1, pk, 0.0), axis=1, keepdims=True)
    t = sh_ref[...] + w0 * a_ref[...] + w1 * b_ref[...]
    mo = 0.1 * t
    out_ref[...] = mo / (jnp.max(jnp.abs(mo), axis=1, keepdims=True) + 1e-6)


def _combine(sh, packed, a, b, T):
    return pl.pallas_call(
        _combine_body,
        grid=(T // TOK,),
        in_specs=[
            pl.BlockSpec((TOK, H), lambda t: (t, 0)),
            pl.BlockSpec((TOK, LANES), lambda t: (t, 0)),
            pl.BlockSpec((TOK, H), lambda t: (t, 0)),
            pl.BlockSpec((TOK, H), lambda t: (t, 0)),
        ],
        out_specs=pl.BlockSpec((TOK, H), lambda t: (t, 0)),
        out_shape=jax.ShapeDtypeStruct((T, H), jnp.float32),
    )(sh, packed, a, b)


def kernel(x, gate_w, moe_bias, sh_up_w, sh_up_b, sh_down_w, sh_down_b,
           ex_up_w, ex_up_b, ex_down_w, ex_down_b):
    B, S, _ = x.shape
    T = B * S
    P = ((T * K + E * (BT - 1)) + BT - 1) // BT * BT
    NT = P // BT
    x2d = x.reshape(T, H)

    gw_pad = jnp.zeros((LANES, H), jnp.float32).at[:E].set(gate_w)
    gb_pad = jnp.zeros((1, LANES), jnp.float32).at[0, :E].set(moe_bias)

    ex_up_bf, ex_down_bf, sh_up_bf, sh_down_bf = _cast_weights(
        ex_up_w, ex_down_w, sh_up_w, sh_down_w)

    packed, po, teo = _router(x2d, gw_pad, gb_pad, T)
    pad_off16 = po[0, :16].astype(jnp.int32)
    te = teo[:NT, 0].astype(jnp.int32)

    sh = _shared(x2d, sh_up_bf, sh_up_b.reshape(1, I), sh_down_bf,
                 sh_down_b.reshape(1, H), T)
    xs = _make_g1(T, P)(pad_off16, packed, x2d)
    ys = _expert_ffn(xs, te, ex_up_bf, ex_up_b, ex_down_bf, ex_down_b, P)
    a, b = _make_g2(T, P)(pad_off16, packed, ys)
    out = _combine(sh, packed, a, b, T)
    return out.reshape(B, S, H)
